# baseline (speedup 1.0000x reference)
import functools

import jax
import jax.numpy as jnp
import numpy as np
from jax import lax
from jax.experimental import pallas as pl
from jax.experimental.pallas import tpu as pltpu
from jax.experimental.pallas import tpu_sc as plsc

_VERT_IDX = np.array([1382, 1522, 1541, 1667, 1493, 428, 179, 1806, 2289,
                      2408, 2405, 2442, 19, 2504, 3016, 3164, 3049, 3060,
                      364, 626, 3454, 3756, 3863, 3844, 3915, 0, 0, 0, 0,
                      0, 0, 0, 2420, 2332, 2131, 2241, 3129, 3133, 2895,
                      3005, 3815, 3778, 3644, 3713, 0, 0], dtype=np.int64)

_B = 4096
_V = 4040
_K = _VERT_IDX.shape[0]
_NROW = 3 * _K
_NT = 23
_RPT = 6

_rows = (np.arange(3, dtype=np.int64)[:, None] * _V
         + _VERT_IDX[None, :]).reshape(-1)
_IDX = np.zeros((_NT, 8), np.int32)
_IDX[:, :_RPT] = _rows.reshape(_NT, _RPT)
_IDX[:, _RPT:] = _IDX[:, :1]


def _row_gather(table_hbm, idx_hbm, out_hbm, idx_v, rows_v, sem):
    wid = lax.axis_index("s") * 2 + lax.axis_index("c")

    @pl.when(wid < _NT)
    def _():
        pltpu.sync_copy(idx_hbm.at[pl.ds(wid, 1)], idx_v)
        pltpu.async_copy(table_hbm.at[idx_v.at[0]], rows_v, sem).wait()
        pltpu.sync_copy(rows_v.at[pl.ds(0, _RPT)],
                        out_hbm.at[pl.ds(wid * _RPT, _RPT)])


def kernel(vertices):
    vt = jnp.transpose(vertices, (2, 1, 0))
    table = vt.reshape(3 * _V, _B)
    idx = jnp.asarray(_IDX)
    mesh = plsc.VectorSubcoreMesh(core_axis_name="c", subcore_axis_name="s")
    run = functools.partial(
        pl.kernel,
        mesh=mesh,
        out_type=jax.ShapeDtypeStruct((_NROW, _B), jnp.float32),
        scratch_types=[
            pltpu.VMEM((1, 8), jnp.int32),
            pltpu.VMEM((8, _B), jnp.float32),
            pltpu.SemaphoreType.DMA,
        ],
        compiler_params=pltpu.CompilerParams(use_tc_tiling_on_sc=False),
    )(_row_gather)
    out = run(table, idx)
    return jnp.transpose(out.reshape(3, _K, _B), (2, 1, 0))


# speedup vs baseline: 47.2735x; 47.2735x over previous
import jax
import jax.numpy as jnp
import numpy as np
from jax.experimental import pallas as pl
from jax.experimental.pallas import tpu as pltpu

_VERT_IDX = np.array([1382, 1522, 1541, 1667, 1493, 428, 179, 1806, 2289,
                      2408, 2405, 2442, 19, 2504, 3016, 3164, 3049, 3060,
                      364, 626, 3454, 3756, 3863, 3844, 3915, 0, 0, 0, 0,
                      0, 0, 0, 2420, 2332, 2131, 2241, 3129, 3133, 2895,
                      3005, 3815, 3778, 3644, 3713, 0, 0], dtype=np.int64)

_B = 4096
_V = 4040
_K = _VERT_IDX.shape[0]


def _gather_body(vt_ref, out_ref, sem):
    def copy(c, k):
        return pltpu.make_async_copy(
            vt_ref.at[pl.ds(c, 1), pl.ds(int(_VERT_IDX[k]), 1), :],
            out_ref.at[pl.ds(c, 1), pl.ds(k, 1), :],
            sem,
        )

    for c in range(3):
        for k in range(_K):
            copy(c, k).start()
    for c in range(3):
        for k in range(_K):
            copy(c, k).wait()


def kernel(vertices):
    vt = jnp.transpose(vertices, (2, 1, 0))
    out_t = pl.pallas_call(
        _gather_body,
        in_specs=[pl.BlockSpec(memory_space=pltpu.MemorySpace.HBM)],
        out_specs=pl.BlockSpec(memory_space=pltpu.MemorySpace.VMEM),
        out_shape=jax.ShapeDtypeStruct((3, _K, _B), jnp.float32),
        scratch_shapes=[pltpu.SemaphoreType.DMA],
    )(vt)
    return jnp.transpose(out_t, (2, 1, 0))


# speedup vs baseline: 51.3948x; 1.0872x over previous
import jax
import jax.numpy as jnp
import numpy as np
from jax.experimental import pallas as pl
from jax.experimental.pallas import tpu as pltpu

_VERT_IDX = np.array([1382, 1522, 1541, 1667, 1493, 428, 179, 1806, 2289,
                      2408, 2405, 2442, 19, 2504, 3016, 3164, 3049, 3060,
                      364, 626, 3454, 3756, 3863, 3844, 3915, 0, 0, 0, 0,
                      0, 0, 0, 2420, 2332, 2131, 2241, 3129, 3133, 2895,
                      3005, 3815, 3778, 3644, 3713, 0, 0], dtype=np.int64)

_B = 4096
_V = 4040
_K = _VERT_IDX.shape[0]


def _gather_body(vt_ref, out_ref, scr_ref, sem_in, sem_out):
    def gather(c, k):
        return pltpu.make_async_copy(
            vt_ref.at[pl.ds(c, 1), pl.ds(int(_VERT_IDX[k]), 1), :],
            scr_ref.at[pl.ds(c, 1), pl.ds(k, 1), :],
            sem_in.at[c],
        )

    def writeback(c):
        return pltpu.make_async_copy(
            scr_ref.at[pl.ds(c, 1)], out_ref.at[pl.ds(c, 1)], sem_out)

    for c in range(3):
        for k in range(_K):
            gather(c, k).start()
    for c in range(3):
        for k in range(_K):
            gather(c, k).wait()
        writeback(c).start()
    for c in range(3):
        writeback(c).wait()


def kernel(vertices):
    vt = jnp.transpose(vertices, (2, 1, 0))
    out_t = pl.pallas_call(
        _gather_body,
        in_specs=[pl.BlockSpec(memory_space=pltpu.MemorySpace.HBM)],
        out_specs=pl.BlockSpec(memory_space=pltpu.MemorySpace.HBM),
        out_shape=jax.ShapeDtypeStruct((3, _K, _B), jnp.float32),
        scratch_shapes=[
            pltpu.VMEM((3, _K, _B), jnp.float32),
            pltpu.SemaphoreType.DMA((3,)),
            pltpu.SemaphoreType.DMA,
        ],
    )(vt)
    return jnp.transpose(out_t, (2, 1, 0))
